# baseline (speedup 1.0000x reference)
import functools

import jax
import jax.numpy as jnp
from jax import lax
from jax.experimental import pallas as pl
from jax.experimental.pallas import tpu as pltpu
from jax.experimental.pallas import tpu_sc as plsc

PADDING_IDX = 1
LANES = 16

_NC = 2
_NS = 16
_NW = _NC * _NS


def _make_sc_bcast(bsz, seq, d):
    cols_pw = seq // _NW
    assert seq % _NW == 0 and cols_pw % LANES == 0 and d % LANES == 0
    ch = 32
    nch = cols_pw // ch
    assert cols_pw % ch == 0
    gpb = cols_pw // LANES
    ngroups = bsz * gpb
    mesh = plsc.VectorSubcoreMesh(core_axis_name="c", subcore_axis_name="s")

    @functools.partial(
        pl.kernel,
        mesh=mesh,
        compiler_params=pltpu.CompilerParams(needs_layout_passes=False),
        out_type=jax.ShapeDtypeStruct((bsz * seq, d), jnp.float32),
        scratch_types=[
            pltpu.VMEM((bsz * cols_pw,), jnp.int32),
            pltpu.VMEM((ch,), jnp.int32),
            pltpu.VMEM((ch,), jnp.int32),
            pltpu.VMEM((ch,), jnp.int32),
            pltpu.VMEM((LANES,), jnp.int32),
            pltpu.VMEM((ch, d), jnp.float32),
            pltpu.VMEM((ch, d), jnp.float32),
            pltpu.VMEM((ch, d), jnp.float32),
            pltpu.VMEM((LANES, d), jnp.float32),
            pltpu.SemaphoreType.DMA,
            pltpu.SemaphoreType.DMA,
            pltpu.SemaphoreType.DMA,
            pltpu.SemaphoreType.DMA,
            pltpu.SemaphoreType.DMA,
            pltpu.SemaphoreType.DMA,
            pltpu.SemaphoreType.DMA,
        ],
    )
    def body(tok_hbm, w_hbm, out_hbm, tok_v, idx0, idx1, idx2, idxg,
             buf0, buf1, buf2, gbuf, sr0, sr1, sr2, sw0, sw1, sw2, sg):
        wid = lax.axis_index("s") * _NC + lax.axis_index("c")
        col0 = wid * cols_pw
        bufs, idxs = (buf0, buf1, buf2), (idx0, idx1, idx2)
        srs, sws = (sr0, sr1, sr2), (sw0, sw1, sw2)
        lane = lax.broadcasted_iota(jnp.int32, (LANES,), 0)

        for b in range(bsz):
            pltpu.sync_copy(tok_hbm.at[pl.ds(b * seq + col0, cols_pw)],
                            tok_v.at[pl.ds(b * cols_pw, cols_pw)])

        nbuf = 3
        writes = {}
        reads = {}

        def start_read(c):
            bi = c % nbuf
            for w in writes.pop(bi, ()):
                w.wait()
            for g in range(ch // LANES):
                idxs[bi][pl.ds(g * LANES, LANES)] = (
                    lane + (col0 + (PADDING_IDX + 1) + c * ch + g * LANES))
            reads[c] = pltpu.async_copy(w_hbm.at[idxs[bi]], bufs[bi], srs[bi])

        for c in range(min(nbuf, nch)):
            start_read(c)
        for c in range(nch):
            bi = c % nbuf
            reads.pop(c).wait()
            writes[bi] = [
                pltpu.async_copy(
                    bufs[bi],
                    out_hbm.at[pl.ds(b * seq + col0 + c * ch, ch)],
                    sws[bi])
                for b in range(bsz)
            ]
            if c + nbuf < nch:
                start_read(c + nbuf)
        for bi in range(nbuf):
            for w in writes.pop(bi, ()):
                w.wait()

        def fix_group(g, carry):
            t = tok_v[pl.ds(g * LANES, LANES)]
            pad = t == PADDING_IDX
            cnt = plsc.all_reduce_population_count(pad)

            @pl.when(cnt[0] > 0)
            def _():
                b = g // gpb
                cbase = (g % gpb) * LANES
                col = lane + (col0 + cbase)
                pos = jnp.where(pad, PADDING_IDX, col + (PADDING_IDX + 1))
                idxg[...] = pos
                pltpu.async_copy(w_hbm.at[idxg], gbuf, sg).wait()
                pltpu.sync_copy(
                    gbuf, out_hbm.at[pl.ds(b * seq + col0 + cbase, LANES)])

            return carry

        lax.fori_loop(0, ngroups, fix_group, 0)

    return body


def kernel(input, weights):
    bsz, seq = input.shape
    _, d = weights.shape
    lookup = _make_sc_bcast(bsz, seq, d)
    out = lookup(input.reshape(-1), weights)
    return out.reshape(bsz, seq, d)


# speedup vs baseline: 1.1204x; 1.1204x over previous
import functools
import math

import jax
import jax.numpy as jnp
import numpy as np
from jax import lax
from jax.experimental import pallas as pl
from jax.experimental.pallas import tpu as pltpu
from jax.experimental.pallas import tpu_sc as plsc

PADDING_IDX = 1
LANES = 16
KBLK = 128

_NC = 2
_NS = 16
_NW = _NC * _NS


def _make_sc_gen(bsz, seq, d):
    half = d // 2
    cols_pw = seq // _NW
    assert seq % _NW == 0 and cols_pw % LANES == 0
    assert d % 2 == 0 and half % KBLK == 0
    nkb = half // KBLK
    vpb = KBLK // LANES
    ch = 32
    nch = cols_pw // ch
    assert cols_pw % ch == 0 and ch > LANES
    gpb = cols_pw // LANES
    ngroups = bsz * gpb
    mesh = plsc.VectorSubcoreMesh(core_axis_name="c", subcore_axis_name="s")

    @functools.partial(
        pl.kernel,
        mesh=mesh,
        compiler_params=pltpu.CompilerParams(needs_layout_passes=False),
        out_type=jax.ShapeDtypeStruct((bsz * seq, d), jnp.float32),
        scratch_types=[
            pltpu.VMEM((bsz * cols_pw,), jnp.int32),
            pltpu.VMEM((d,), jnp.float32),
            pltpu.VMEM((LANES,), jnp.int32),
            pltpu.VMEM((ch, d), jnp.float32),
            pltpu.VMEM((ch, d), jnp.float32),
            pltpu.VMEM((LANES, d), jnp.float32),
            pltpu.SemaphoreType.DMA,
            pltpu.SemaphoreType.DMA,
            pltpu.SemaphoreType.DMA,
        ],
    )
    def body(tok_hbm, w_hbm, rot_hbm, out_hbm, tok_v, rot_v, idxg,
             buf0, buf1, gbuf, sg, sw0, sw1):
        wid = lax.axis_index("s") * _NC + lax.axis_index("c")
        col0 = wid * cols_pw
        p0 = col0 + (PADDING_IDX + 1)
        bufs, sws = (buf0, buf1), (sw0, sw1)
        lane = lax.broadcasted_iota(jnp.int32, (LANES,), 0)

        for b in range(bsz):
            pltpu.sync_copy(tok_hbm.at[pl.ds(b * seq + col0, cols_pw)],
                            tok_v.at[pl.ds(b * cols_pw, cols_pw)])
        pltpu.sync_copy(rot_hbm, rot_v)

        idxg[...] = lane + p0
        pltpu.async_copy(w_hbm.at[idxg], buf0.at[pl.ds(0, LANES)], sg).wait()

        rots = [[rot_v[pl.ds(kb * KBLK + v * LANES, LANES)]
                 for v in range(vpb)] for kb in range(nkb)]
        rotc = [[rot_v[pl.ds(half + kb * KBLK + v * LANES, LANES)]
                 for v in range(vpb)] for kb in range(nkb)]
        state = [([buf0[LANES - 1, pl.ds(kb * KBLK + v * LANES, LANES)]
                   for v in range(vpb)],
                  [buf0[LANES - 1, pl.ds(half + kb * KBLK + v * LANES, LANES)]
                   for v in range(vpb)]) for kb in range(nkb)]

        def gen_rows(buf, r_lo, r_hi, state):
            new_state = []
            for kb in range(nkb):
                sv, cv = state[kb]

                def row_body(r, carry, kb=kb, buf=buf):
                    sv, cv, sf, cf = carry
                    ns, nc = [], []
                    for v in range(vpb):
                        ns.append(sv[v] * cf[v] + cv[v] * sf[v])
                        nc.append(cv[v] * cf[v] - sv[v] * sf[v])
                        buf[r, pl.ds(kb * KBLK + v * LANES, LANES)] = ns[v]
                        buf[r, pl.ds(half + kb * KBLK + v * LANES, LANES)] = (
                            nc[v])
                    return tuple(ns), tuple(nc), sf, cf

                sv, cv, _, _ = lax.fori_loop(
                    r_lo, r_hi, row_body,
                    (tuple(sv), tuple(cv), tuple(rots[kb]), tuple(rotc[kb])))
                new_state.append((sv, cv))
            return new_state

        writes = {}
        for c in range(nch):
            bi = c % 2
            for w in writes.pop(bi, ()):
                w.wait()
            state = gen_rows(bufs[bi], LANES if c == 0 else 0, ch, state)
            writes[bi] = [
                pltpu.async_copy(
                    bufs[bi],
                    out_hbm.at[pl.ds(b * seq + col0 + c * ch, ch)],
                    sws[bi])
                for b in range(bsz)
            ]
        for bi in (0, 1):
            for w in writes.pop(bi, ()):
                w.wait()

        def fix_group(g, carry):
            t = tok_v[pl.ds(g * LANES, LANES)]
            pad = t == PADDING_IDX
            cnt = plsc.all_reduce_population_count(pad)

            @pl.when(cnt[0] > 0)
            def _():
                b = g // gpb
                cbase = (g % gpb) * LANES
                col = lane + (col0 + cbase)
                pos = jnp.where(pad, PADDING_IDX, col + (PADDING_IDX + 1))
                idxg[...] = pos
                pltpu.async_copy(w_hbm.at[idxg], gbuf, sg).wait()
                pltpu.sync_copy(
                    gbuf, out_hbm.at[pl.ds(b * seq + col0 + cbase, LANES)])

            return carry

        lax.fori_loop(0, ngroups, fix_group, 0)

    return body


def kernel(input, weights):
    bsz, seq = input.shape
    _, d = weights.shape
    half = d // 2
    freq = np.exp(np.arange(half, dtype=np.float64)
                  * -(math.log(10000.0) / (half - 1)))
    rot = np.concatenate([np.sin(freq), np.cos(freq)]).astype(np.float32)
    lookup = _make_sc_gen(bsz, seq, d)
    out = lookup(input.reshape(-1), weights, jnp.asarray(rot))
    return out.reshape(bsz, seq, d)
